# baseline (speedup 1.0000x reference)
import functools

import jax
import jax.numpy as jnp
from jax.experimental import pallas as pl
from jax.experimental.pallas import tpu as pltpu

B, N, C, K = 2, 2048, 1024, 32


def _fc_body(x_ref, wt_ref, bias_ref, feat_ref, pos_ref):
    fp = jax.lax.dot_general(
        x_ref[...], wt_ref[...], (((1,), (0,)), ((), ())),
        preferred_element_type=jnp.float32,
    ) + bias_ref[...]
    feat_ref[...] = fp[:, :C]
    pr = fp[:, C:]
    nrm = jnp.sqrt(jnp.sum(pr * pr, axis=1, keepdims=True))
    pos_ref[...] = pr / jnp.maximum(nrm, 1e-12)


def _attn_body(posb_ref, posf_ref, feat_ref, out_ref, sim_ref, a_ref,
               *, bm: int, gr: int):
    pb = posb_ref[0]
    pf = posf_ref[0]
    sim_ref[...] = jax.lax.dot_general(
        pb, pf, (((1,), (1,)), ((), ())),
        preferred_element_type=jnp.float32,
    )

    def group(g, _):
        s0 = sim_ref[pl.ds(g * gr, gr), :]
        rowmax = jnp.max(s0, axis=1, keepdims=True)

        def step(_, s):
            m = jnp.max(s, axis=1, keepdims=True)
            return jnp.where(s == m, -jnp.inf, s)

        s_fin = jax.lax.fori_loop(0, K, step, s0, unroll=True)
        e = jnp.where(s_fin == -jnp.inf, jnp.exp(s0 - rowmax), 0.0)
        a_ref[pl.ds(g * gr, gr), :] = e / jnp.sum(e, axis=1, keepdims=True)
        return 0

    jax.lax.fori_loop(0, bm // gr, group, 0)

    out_ref[0] = jax.lax.dot_general(
        a_ref[...], feat_ref[0], (((1,), (0,)), ((), ())),
        preferred_element_type=jnp.float32,
    )


@jax.jit
def kernel(x, W, bias):
    bm = 256
    gr = 32
    x2 = x.reshape(B * N, C)
    wt = W.T
    bias2 = bias.reshape(1, 2 * C)

    feat, pos = pl.pallas_call(
        _fc_body,
        grid=(B * N // bm,),
        in_specs=[
            pl.BlockSpec((bm, C), lambda i: (i, 0)),
            pl.BlockSpec((C, 2 * C), lambda i: (0, 0)),
            pl.BlockSpec((1, 2 * C), lambda i: (0, 0)),
        ],
        out_specs=[
            pl.BlockSpec((bm, C), lambda i: (i, 0)),
            pl.BlockSpec((bm, C), lambda i: (i, 0)),
        ],
        out_shape=[
            jax.ShapeDtypeStruct((B * N, C), jnp.float32),
            jax.ShapeDtypeStruct((B * N, C), jnp.float32),
        ],
    )(x2, wt, bias2)

    feat3 = feat.reshape(B, N, C)
    pos3 = pos.reshape(B, N, C)

    out = pl.pallas_call(
        functools.partial(_attn_body, bm=bm, gr=gr),
        grid=(B, N // bm),
        in_specs=[
            pl.BlockSpec((1, bm, C), lambda b, i: (b, i, 0)),
            pl.BlockSpec((1, N, C), lambda b, i: (b, 0, 0)),
            pl.BlockSpec((1, N, C), lambda b, i: (b, 0, 0)),
        ],
        out_specs=pl.BlockSpec((1, bm, C), lambda b, i: (b, i, 0)),
        out_shape=jax.ShapeDtypeStruct((B, N, C), jnp.float32),
        scratch_shapes=[
            pltpu.VMEM((bm, N), jnp.float32),
            pltpu.VMEM((bm, N), jnp.float32),
        ],
    )(pos3, pos3, feat3)

    return out


# speedup vs baseline: 2.0447x; 2.0447x over previous
import functools

import jax
import jax.numpy as jnp
from jax.experimental import pallas as pl

B, N, C, K = 2, 2048, 1024, 32


def _fc_body(x_ref, wt_ref, bias_ref, feat_ref, pos_ref):
    fp = jax.lax.dot_general(
        x_ref[...], wt_ref[...], (((1,), (0,)), ((), ())),
        preferred_element_type=jnp.float32,
    ) + bias_ref[...]
    feat_ref[...] = fp[:, :C]
    pr = fp[:, C:]
    nrm = jnp.sqrt(jnp.sum(pr * pr, axis=1, keepdims=True))
    pos_ref[...] = pr / jnp.maximum(nrm, 1e-12)


def _attn_body(posb_ref, posf_ref, feat_ref, out_ref, *, bm: int):
    pb = posb_ref[0]
    pf = posf_ref[0]
    sim = jax.lax.dot_general(
        pb, pf, (((1,), (1,)), ((), ())),
        preferred_element_type=jnp.float32,
    )
    rowmax = jnp.max(sim, axis=1, keepdims=True)

    def step(_, s):
        m = jnp.max(s, axis=1, keepdims=True)
        return jnp.where(s == m, -jnp.inf, s)

    s_fin = jax.lax.fori_loop(0, K, step, sim, unroll=8)
    mask = s_fin == -jnp.inf

    e = jnp.where(mask, jnp.exp(sim - rowmax), 0.0)
    a = e / jnp.sum(e, axis=1, keepdims=True)
    out_ref[0] = jax.lax.dot_general(
        a, feat_ref[0], (((1,), (0,)), ((), ())),
        preferred_element_type=jnp.float32,
    )


@jax.jit
def kernel(x, W, bias):
    bm = 256
    x2 = x.reshape(B * N, C)
    wt = W.T
    bias2 = bias.reshape(1, 2 * C)

    feat, pos = pl.pallas_call(
        _fc_body,
        grid=(B * N // bm,),
        in_specs=[
            pl.BlockSpec((bm, C), lambda i: (i, 0)),
            pl.BlockSpec((C, 2 * C), lambda i: (0, 0)),
            pl.BlockSpec((1, 2 * C), lambda i: (0, 0)),
        ],
        out_specs=[
            pl.BlockSpec((bm, C), lambda i: (i, 0)),
            pl.BlockSpec((bm, C), lambda i: (i, 0)),
        ],
        out_shape=[
            jax.ShapeDtypeStruct((B * N, C), jnp.float32),
            jax.ShapeDtypeStruct((B * N, C), jnp.float32),
        ],
    )(x2, wt, bias2)

    feat3 = feat.reshape(B, N, C)
    pos3 = pos.reshape(B, N, C)

    out = pl.pallas_call(
        functools.partial(_attn_body, bm=bm),
        grid=(B, N // bm),
        in_specs=[
            pl.BlockSpec((1, bm, C), lambda b, i: (b, i, 0)),
            pl.BlockSpec((1, N, C), lambda b, i: (b, 0, 0)),
            pl.BlockSpec((1, N, C), lambda b, i: (b, 0, 0)),
        ],
        out_specs=pl.BlockSpec((1, bm, C), lambda b, i: (b, i, 0)),
        out_shape=jax.ShapeDtypeStruct((B, N, C), jnp.float32),
    )(pos3, pos3, feat3)

    return out


# speedup vs baseline: 2.3319x; 1.1404x over previous
import functools

import jax
import jax.numpy as jnp
from jax.experimental import pallas as pl

B, N, C, K = 2, 2048, 1024, 32


def _fc_body(x_ref, wt_ref, bias_ref, feat_ref, pos_ref):
    fp = jax.lax.dot_general(
        x_ref[...], wt_ref[...], (((1,), (0,)), ((), ())),
        preferred_element_type=jnp.float32,
    ) + bias_ref[...]
    feat_ref[...] = fp[:, :C]
    pr = fp[:, C:]
    nrm = jnp.sqrt(jnp.sum(pr * pr, axis=1, keepdims=True))
    pos_ref[...] = pr / jnp.maximum(nrm, 1e-12)


def _attn_body(posb_ref, posf_ref, feat_ref, out_ref, *, bm: int):
    pb = posb_ref[0]
    pf = posf_ref[0]
    sim = jax.lax.dot_general(
        pb, pf, (((1,), (1,)), ((), ())),
        preferred_element_type=jnp.float32,
    )
    rowmax = jnp.max(sim, axis=1, keepdims=True)

    def step(_, s):
        m = jnp.max(s, axis=1, keepdims=True)
        return jnp.where(s == m, -jnp.inf, s)

    s_fin = jax.lax.fori_loop(0, K, step, sim, unroll=K)
    mask = s_fin == -jnp.inf

    e = jnp.where(mask, jnp.exp(sim - rowmax), 0.0)
    a = e / jnp.sum(e, axis=1, keepdims=True)
    out_ref[0] = jax.lax.dot_general(
        a, feat_ref[0], (((1,), (0,)), ((), ())),
        preferred_element_type=jnp.float32,
    )


@jax.jit
def kernel(x, W, bias):
    bm = 256
    x2 = x.reshape(B * N, C)
    wt = W.T
    bias2 = bias.reshape(1, 2 * C)

    feat, pos = pl.pallas_call(
        _fc_body,
        grid=(B * N // bm,),
        in_specs=[
            pl.BlockSpec((bm, C), lambda i: (i, 0)),
            pl.BlockSpec((C, 2 * C), lambda i: (0, 0)),
            pl.BlockSpec((1, 2 * C), lambda i: (0, 0)),
        ],
        out_specs=[
            pl.BlockSpec((bm, C), lambda i: (i, 0)),
            pl.BlockSpec((bm, C), lambda i: (i, 0)),
        ],
        out_shape=[
            jax.ShapeDtypeStruct((B * N, C), jnp.float32),
            jax.ShapeDtypeStruct((B * N, C), jnp.float32),
        ],
    )(x2, wt, bias2)

    feat3 = feat.reshape(B, N, C)
    pos3 = pos.reshape(B, N, C)

    out = pl.pallas_call(
        functools.partial(_attn_body, bm=bm),
        grid=(B, N // bm),
        in_specs=[
            pl.BlockSpec((1, bm, C), lambda b, i: (b, i, 0)),
            pl.BlockSpec((1, N, C), lambda b, i: (b, 0, 0)),
            pl.BlockSpec((1, N, C), lambda b, i: (b, 0, 0)),
        ],
        out_specs=pl.BlockSpec((1, bm, C), lambda b, i: (b, i, 0)),
        out_shape=jax.ShapeDtypeStruct((B, N, C), jnp.float32),
    )(pos3, pos3, feat3)

    return out


# speedup vs baseline: 2.4310x; 1.0425x over previous
import functools

import jax
import jax.numpy as jnp
from jax.experimental import pallas as pl

B, N, C, K = 2, 2048, 1024, 32


def _fc_body(x_ref, wt_ref, bias_ref, feat_ref, pos_ref):
    fp = jax.lax.dot_general(
        x_ref[...], wt_ref[...], (((1,), (0,)), ((), ())),
        preferred_element_type=jnp.float32,
    ) + bias_ref[...]
    feat_ref[...] = fp[:, :C].astype(jnp.bfloat16)
    pr = fp[:, C:]
    nrm = jnp.sqrt(jnp.sum(pr * pr, axis=1, keepdims=True))
    pos_ref[...] = (pr / jnp.maximum(nrm, 1e-12)).astype(jnp.bfloat16)


def _attn_body(posb_ref, posf_ref, feat_ref, out_ref, *, bm: int):
    pb = posb_ref[0]
    pf = posf_ref[0]
    sim = jax.lax.dot_general(
        pb, pf, (((1,), (1,)), ((), ())),
        preferred_element_type=jnp.float32,
    )

    def step(_, s):
        m = jnp.max(s, axis=1, keepdims=True)
        return jnp.where(s == m, -jnp.inf, s)

    hm = bm // 2
    for h in range(2):
        sh = sim[h * hm:(h + 1) * hm]
        s_fin = jax.lax.fori_loop(0, K, step, sh, unroll=K)
        e = jnp.where(s_fin == -jnp.inf, jnp.exp(sh), 0.0)
        r = 1.0 / jnp.sum(e, axis=1, keepdims=True)
        o = jax.lax.dot_general(
            e, feat_ref[0], (((1,), (0,)), ((), ())),
            preferred_element_type=jnp.float32,
        )
        out_ref[0, h * hm:(h + 1) * hm, :] = o * r


@jax.jit
def kernel(x, W, bias):
    bm = 256
    x2 = x.reshape(B * N, C)
    wt = W.T
    bias2 = bias.reshape(1, 2 * C)

    feat, pos = pl.pallas_call(
        _fc_body,
        grid=(B * N // bm,),
        in_specs=[
            pl.BlockSpec((bm, C), lambda i: (i, 0)),
            pl.BlockSpec((C, 2 * C), lambda i: (0, 0)),
            pl.BlockSpec((1, 2 * C), lambda i: (0, 0)),
        ],
        out_specs=[
            pl.BlockSpec((bm, C), lambda i: (i, 0)),
            pl.BlockSpec((bm, C), lambda i: (i, 0)),
        ],
        out_shape=[
            jax.ShapeDtypeStruct((B * N, C), jnp.bfloat16),
            jax.ShapeDtypeStruct((B * N, C), jnp.bfloat16),
        ],
    )(x2, wt, bias2)

    feat3 = feat.reshape(B, N, C)
    pos3 = pos.reshape(B, N, C)

    out = pl.pallas_call(
        functools.partial(_attn_body, bm=bm),
        grid=(B, N // bm),
        in_specs=[
            pl.BlockSpec((1, bm, C), lambda b, i: (b, i, 0)),
            pl.BlockSpec((1, N, C), lambda b, i: (b, 0, 0)),
            pl.BlockSpec((1, N, C), lambda b, i: (b, 0, 0)),
        ],
        out_specs=pl.BlockSpec((1, bm, C), lambda b, i: (b, i, 0)),
        out_shape=jax.ShapeDtypeStruct((B, N, C), jnp.float32),
    )(pos3, pos3, feat3)

    return out


# speedup vs baseline: 2.9160x; 1.1995x over previous
import functools

import jax
import jax.numpy as jnp
from jax.experimental import pallas as pl

B, N, C, K = 2, 2048, 1024, 32


def _fc_body(x_ref, wt_ref, bias_ref, feat_ref, pos_ref):
    fp = jax.lax.dot_general(
        x_ref[...], wt_ref[...], (((1,), (1,)), ((), ())),
        preferred_element_type=jnp.float32,
    ) + bias_ref[...]
    feat_ref[...] = fp[:, :C].astype(jnp.bfloat16)
    pr = fp[:, C:]
    nrm = jnp.sqrt(jnp.sum(pr * pr, axis=1, keepdims=True))
    pos_ref[...] = (pr / jnp.maximum(nrm, 1e-12)).astype(jnp.bfloat16)


def _attn_body(posb_ref, posf_ref, feat_ref, out_ref, *, bm: int):
    pb = posb_ref[0]
    pf = posf_ref[0]
    sim = jax.lax.dot_general(
        pb, pf, (((1,), (1,)), ((), ())),
        preferred_element_type=jnp.float32,
    )

    def step(_, s):
        m = jnp.max(s, axis=1, keepdims=True)
        return jnp.where(s == m, -jnp.inf, s)

    hm = bm // 2
    for h in range(2):
        sh = sim[h * hm:(h + 1) * hm]
        s_fin = jax.lax.fori_loop(0, K, step, sh, unroll=K)
        e = jnp.where(s_fin == -jnp.inf, jnp.exp(sh), 0.0)
        r = 1.0 / jnp.sum(e, axis=1, keepdims=True)
        o = jax.lax.dot_general(
            e, feat_ref[0], (((1,), (0,)), ((), ())),
            preferred_element_type=jnp.float32,
        )
        out_ref[0, h * hm:(h + 1) * hm, :] = o * r


@jax.jit
def kernel(x, W, bias):
    bm = 512
    x2 = x.reshape(B * N, C)
    wt = W
    bias2 = bias.reshape(1, 2 * C)

    feat, pos = pl.pallas_call(
        _fc_body,
        grid=(B * N // bm,),
        in_specs=[
            pl.BlockSpec((bm, C), lambda i: (i, 0)),
            pl.BlockSpec((2 * C, C), lambda i: (0, 0)),
            pl.BlockSpec((1, 2 * C), lambda i: (0, 0)),
        ],
        out_specs=[
            pl.BlockSpec((bm, C), lambda i: (i, 0)),
            pl.BlockSpec((bm, C), lambda i: (i, 0)),
        ],
        out_shape=[
            jax.ShapeDtypeStruct((B * N, C), jnp.bfloat16),
            jax.ShapeDtypeStruct((B * N, C), jnp.bfloat16),
        ],
    )(x2, wt, bias2)

    feat3 = feat.reshape(B, N, C)
    pos3 = pos.reshape(B, N, C)

    out = pl.pallas_call(
        functools.partial(_attn_body, bm=bm),
        grid=(B, N // bm),
        in_specs=[
            pl.BlockSpec((1, bm, C), lambda b, i: (b, i, 0)),
            pl.BlockSpec((1, N, C), lambda b, i: (b, 0, 0)),
            pl.BlockSpec((1, N, C), lambda b, i: (b, 0, 0)),
        ],
        out_specs=pl.BlockSpec((1, bm, C), lambda b, i: (b, i, 0)),
        out_shape=jax.ShapeDtypeStruct((B, N, C), jnp.float32),
    )(pos3, pos3, feat3)

    return out
